# baseline (speedup 1.0000x reference)
import functools

import jax
import jax.numpy as jnp
from jax import lax
from jax.experimental import pallas as pl
from jax.experimental.pallas import tpu as pltpu
from jax.experimental.pallas import tpu_sc as plsc

NC = 2
NS = 16
K = 80


def _sc_agg(x, eflat, z2, z1, NP, with_counts):
    N, D = x.shape
    NW = NC * NS
    E = eflat.shape[0] // 2
    ept = E // NW
    n_chunks = ept // K
    rpt = NP // NS
    assert E % NW == 0 and ept % K == 0 and rpt % 8 == 0
    assert z2.shape == (rpt, D) and z1.shape == (NP,)
    assert n_chunks % 3 == 2 and n_chunks >= 8

    mesh = plsc.VectorSubcoreMesh(core_axis_name="c", subcore_axis_name="s")

    out_type = [jax.ShapeDtypeStruct((NC, NP, D), jnp.float32)]
    if with_counts:
        out_type += [jax.ShapeDtypeStruct((NP,), jnp.float32),
                     jax.ShapeDtypeStruct((NP,), jnp.float32)]

    @functools.partial(
        pl.kernel, mesh=mesh, out_type=out_type,
        scratch_types=[
            pltpu.VMEM((ept,), jnp.int32),
            pltpu.VMEM((K,), jnp.int32),
            pltpu.VMEM((K,), jnp.int32),
            pltpu.VMEM((K,), jnp.int32),
            pltpu.VMEM((K, D), jnp.float32),
            pltpu.VMEM((K, D), jnp.float32),
            pltpu.VMEM((K, D), jnp.float32),
            pltpu.VMEM((K,), jnp.float32),
            pltpu.VMEM_SHARED((NP, D), jnp.float32),
            pltpu.VMEM_SHARED((NP,), jnp.float32),
        ] + [pltpu.SemaphoreType.DMA] * 12,
    )
    def agg(*refs):
        if with_counts:
            (x_hbm, e_hbm, z2_hbm, z1_hbm,
             out_hbm, c0_hbm, c1_hbm, sidx,
             dx0, dx1, dx2, rw0, rw1, rw2, ones, acc, cacc, *sems) = refs
        else:
            (x_hbm, e_hbm, z2_hbm, z1_hbm, out_hbm, sidx,
             dx0, dx1, dx2, rw0, rw1, rw2, ones, acc, cacc, *sems) = refs
        didxs = [dx0, dx1, dx2]
        rows = [rw0, rw1, rw2]
        gsem = sems[0:3]
        ssem = sems[3:6]
        isem = sems[6:9]
        csem = sems[9:12]
        c = lax.axis_index("c")
        s = lax.axis_index("s")
        wid = c * NS + s
        ebase = wid * ept
        dbase = E + wid * ept

        zcp = pltpu.async_copy(z2_hbm, acc.at[pl.ds(s * rpt, rpt)], gsem[0])
        scp = pltpu.async_copy(e_hbm.at[pl.ds(ebase, ept)], sidx, gsem[1])
        if with_counts:
            @pl.when(s == 0)
            def _():
                pltpu.sync_copy(z1_hbm, cacc)

            for j in range(K // 16):
                ones[pl.ds(j * 16, 16)] = jnp.ones((16,), jnp.float32)
        zcp.wait()
        scp.wait()

        plsc.subcore_barrier()

        def ifire(ci, b):
            pltpu.async_copy(e_hbm.at[pl.ds(dbase + ci * K, K)],
                             didxs[b], isem[b])

        def idrain(b):
            pltpu.make_async_copy(e_hbm.at[pl.ds(0, K)],
                                  didxs[b], isem[b]).wait()

        def gfire(ci, b):
            pltpu.async_copy(x_hbm.at[sidx.at[pl.ds(ci * K, K)]],
                             rows[b], gsem[b])

        def gdrain(b):
            pltpu.make_async_copy(x_hbm.at[sidx.at[pl.ds(0, K)]],
                                  rows[b], gsem[b]).wait()

        def sfire(b):
            pltpu.async_copy(rows[b], acc.at[didxs[b]], ssem[b], add=True)

        def sdrain(b):
            pltpu.make_async_copy(rows[b], acc.at[didxs[b]], ssem[b]).wait()

        def cfire(b):
            pltpu.async_copy(ones, cacc.at[didxs[b]], csem[b], add=True)

        def cdrain(b):
            pltpu.make_async_copy(ones, cacc.at[didxs[b]], csem[b]).wait()

        def proc(ci, b, drain_prev=True, fire_next=True):
            bn = (b + 1) % 3
            if drain_prev:
                sdrain(bn)
                if with_counts:
                    cdrain(bn)
            if fire_next:
                ifire(ci + 1, bn)
                gfire(ci + 1, bn)
            gdrain(b)
            idrain(b)
            sfire(b)
            if with_counts:
                cfire(b)

        ifire(0, 0)
        gfire(0, 0)
        proc(0, 0, drain_prev=False)
        proc(1, 1, drain_prev=False)
        proc(2, 2)

        def triple(t, carry):
            c0 = 3 * t
            proc(c0, 0)
            proc(c0 + 1, 1)
            proc(c0 + 2, 2)
            return carry

        lax.fori_loop(1, (n_chunks - 8) // 3 + 1, triple, 0)

        proc(n_chunks - 5, 0)
        proc(n_chunks - 4, 1)
        proc(n_chunks - 3, 2)
        proc(n_chunks - 2, 0)
        proc(n_chunks - 1, 1, fire_next=False)
        for b in (0, 1):
            sdrain(b)
            if with_counts:
                cdrain(b)

        plsc.subcore_barrier()

        pltpu.sync_copy(acc.at[pl.ds(s * rpt, rpt)],
                        out_hbm.at[c, pl.ds(s * rpt, rpt)])

        if with_counts:
            @pl.when(jnp.logical_and(s == 0, c == 0))
            def _():
                pltpu.sync_copy(cacc, c0_hbm)

            @pl.when(jnp.logical_and(s == 0, c == 1))
            def _():
                pltpu.sync_copy(cacc, c1_hbm)

    res = agg(x, eflat, z2, z1)
    if not with_counts and isinstance(res, (list, tuple)):
        return res[0]
    return res


def _tc_self(x, wr, b2d):
    N, D = x.shape
    BN = 2048
    grid = (N + BN - 1) // BN

    def body(x_ref, wr_ref, b_ref, o_ref):
        dn = (((1,), (1,)), ((), ()))
        o_ref[...] = lax.dot_general(
            x_ref[...], wr_ref[...], dn,
            preferred_element_type=jnp.float32) + b_ref[...]

    return pl.pallas_call(
        body,
        grid=(grid,),
        in_specs=[
            pl.BlockSpec((BN, D), lambda i: (i, 0)),
            pl.BlockSpec((D, D), lambda i: (0, 0)),
            pl.BlockSpec((1, D), lambda i: (0, 0)),
        ],
        out_specs=pl.BlockSpec((BN, D), lambda i: (i, 0)),
        out_shape=jax.ShapeDtypeStruct((N, D), jnp.float32),
    )(x, wr, b2d)


def _tc_mean(parts, cnt0, cnt1, sp, wl, relu, N):
    NP = parts.shape[1]
    D = parts.shape[2]
    BN = 2048
    grid = (N + BN - 1) // BN
    assert grid * BN <= NP

    def body(part_ref, c0_ref, c1_ref, sp_ref, wl_ref, o_ref):
        i = pl.program_id(0)
        csum = c0_ref[pl.ds(i * BN, BN)] + c1_ref[pl.ds(i * BN, BN)]
        inv = 1.0 / jnp.maximum(csum, 1.0)
        agg = part_ref[0] + part_ref[1]
        mean = agg * inv[:, None]
        dn = (((1,), (1,)), ((), ()))
        h = lax.dot_general(mean, wl_ref[...], dn,
                            preferred_element_type=jnp.float32) + sp_ref[...]
        if relu:
            h = jnp.maximum(h, 0.0)
        o_ref[...] = h

    return pl.pallas_call(
        body,
        grid=(grid,),
        in_specs=[
            pl.BlockSpec((2, BN, D), lambda i: (0, i, 0)),
            pl.BlockSpec((NP,), lambda i: (0,)),
            pl.BlockSpec((NP,), lambda i: (0,)),
            pl.BlockSpec((BN, D), lambda i: (i, 0)),
            pl.BlockSpec((D, D), lambda i: (0, 0)),
        ],
        out_specs=pl.BlockSpec((BN, D), lambda i: (i, 0)),
        out_shape=jax.ShapeDtypeStruct((N, D), jnp.float32),
    )(parts, cnt0, cnt1, sp, wl)


@jax.jit
def kernel(x, edge_index, W1l, b1, W1r, W2l, b2, W2r):
    N, D = x.shape
    NP = ((N + 2047) // 2048) * 2048
    E = edge_index.shape[1]
    eflat = edge_index.reshape(2 * E)
    z2 = jnp.zeros((NP // NS, D), jnp.float32)
    z1 = jnp.zeros((NP,), jnp.float32)

    sp1 = _tc_self(x, W1r, b1.reshape(1, D))
    p1, c1a, c1b = _sc_agg(x, eflat, z2, z1, NP, with_counts=True)
    h = _tc_mean(p1, c1a, c1b, sp1, W1l, True, N)
    sp2 = _tc_self(h, W2r, b2.reshape(1, D))
    p2 = _sc_agg(h, eflat, z2, z1, NP, with_counts=False)
    out = _tc_mean(p2, c1a, c1b, sp2, W2l, False, N)
    return out
